# baseline (speedup 1.0000x reference)
import jax
import jax.numpy as jnp
from jax import lax
from jax.experimental import pallas as pl
from jax.experimental.pallas import tpu as pltpu
from jax.experimental.pallas import tpu_sc as plsc

MBITS = 7
SHIFT = 23 - MBITS
LO = 111 << MBITS
NBINS = 24 << MBITS
NCOLS = 2 * NBINS
RSTRIDE = NCOLS + 1
NLANES = 16
HIST_WORDS = NLANES * RSTRIDE
N_IMG = 8
IMG_ELEMS = 512 * 512
NWORKERS = 32
PER_TILE = N_IMG * IMG_ELEMS // NWORKERS
CHUNK = 8192
NCHUNKS = PER_TILE // CHUNK


ROWS_PER_CHUNK = 8
NROWCHUNKS = 128 // ROWS_PER_CHUNK


def _sc_hist(pred_hbm, targ_hbm, out_hbm, hist, pbufs, tbufs, obuf,
             sem_p0, sem_t0, sem_p1, sem_t1):
    wid = lax.axis_index("s") * 2 + lax.axis_index("c")
    img = wid >> 2
    r0 = (wid & 3) * 128
    lane = lax.iota(jnp.int32, NLANES)
    laneoff = lane * RSTRIDE
    ones = jnp.ones((NLANES,), jnp.float32)
    zeros = jnp.zeros((NLANES,), jnp.float32)

    @plsc.parallel_loop(0, HIST_WORDS // 128, unroll=2)
    def _zero(j):
        b = j * 128
        for k in range(8):
            hist[pl.ds(b + k * NLANES, NLANES)] = zeros

    hist[pl.ds(HIST_WORDS - NLANES, NLANES)] = zeros

    sems = [[sem_p0, sem_t0], [sem_p1, sem_t1]]
    cps = {}

    def issue(c):
        slot = c & 1
        rs = r0 + c * ROWS_PER_CHUNK
        cps[c] = (
            pltpu.async_copy(pred_hbm.at[img, pl.ds(rs, ROWS_PER_CHUNK), :],
                             pbufs.at[slot], sems[slot][0]),
            pltpu.async_copy(targ_hbm.at[img, pl.ds(rs, ROWS_PER_CHUNK), :],
                             tbufs.at[slot], sems[slot][1]),
        )

    issue(0)
    for c in range(NROWCHUNKS):
        slot = c & 1
        cps[c][0].wait()
        cps[c][1].wait()
        if c + 1 < NROWCHUNKS:
            issue(c + 1)

        def body(j, carry):
            r = j >> 1
            cb = (j & 1) * 256
            for k in range(16):
                off = cb + k * NLANES
                p = pbufs[slot, r, pl.ds(off, NLANES)]
                t = tbufs[slot, r, pl.ds(off, NLANES)]
                e = jnp.where(t > 0, 1.0 - p, 1.0 + p)
                bits = lax.bitcast_convert_type(e, jnp.int32)
                b = jnp.minimum(jnp.maximum((bits >> SHIFT) - LO, 0), NBINS - 1)
                col = b + t * NBINS
                plsc.addupdate_scatter(hist, [laneoff + col], ones)
            return carry

        lax.fori_loop(0, ROWS_PER_CHUNK * 512 // 256, body, 0)

    @plsc.parallel_loop(0, NCOLS // 32, unroll=2)
    def _reduce(j):
        for k2 in range(2):
            cbase = j * 32 + k2 * NLANES
            acc = hist[pl.ds(cbase, NLANES)]
            for r in range(1, NLANES):
                acc = acc + hist[pl.ds(r * RSTRIDE + cbase, NLANES)]
            obuf[cbase >> 7, pl.ds(cbase & 127, NLANES)] = acc

    pltpu.sync_copy(obuf, out_hbm.at[wid])


def _cumsum(x, axis):
    n = x.shape[axis]
    k = 1
    while k < n:
        zshape = list(x.shape)
        zshape[axis] = k
        shifted = jnp.concatenate(
            [jnp.zeros(zshape, x.dtype), lax.slice_in_dim(x, 0, n - k, axis=axis)],
            axis=axis,
        )
        x = x + shifted
        k *= 2
    return x


def _tc_loss(hist_ref, out_ref):
    i = pl.program_id(0)
    h = jnp.sum(hist_ref[...], axis=0)
    m = h[0:24, :]
    p = h[24:48, :]
    sp = jnp.sum(p)
    sm = jnp.sum(m)
    rowm = jnp.sum(m, axis=1, keepdims=True)
    rowp = jnp.sum(p, axis=1, keepdims=True)
    pm = _cumsum(m, 1) + (_cumsum(rowm, 0) - rowm)
    pp = _cumsum(p, 1) + (_cumsum(rowp, 0) - rowp)
    a = sp - pp
    b = sm - pm
    r = lax.broadcasted_iota(jnp.int32, (24, 128), 0)
    c = lax.broadcasted_iota(jnp.int32, (24, 128), 1)
    vbits = ((r * 128 + c + LO) << SHIFT) + (1 << (SHIFT - 1))
    v = lax.bitcast_convert_type(vbits, jnp.float32)
    gu = (a + b + p + m) / jnp.maximum(sp + b + m, 1.0)
    gl = (a + b) / jnp.maximum(sp + b, 1.0)
    li = jnp.sum(v * (gu - gl))

    @pl.when(i == 0)
    def _():
        out_ref[...] = jnp.zeros((1, 1), jnp.float32)

    out_ref[...] = out_ref[...] + li / N_IMG


def kernel(pred, target):
    sc = pl.kernel(
        _sc_hist,
        out_type=jax.ShapeDtypeStruct((NWORKERS, 48, 128), jnp.float32),
        mesh=plsc.VectorSubcoreMesh(core_axis_name="c", subcore_axis_name="s"),
        compiler_params=pltpu.CompilerParams(needs_layout_passes=False),
        scratch_types=[
            pltpu.VMEM((HIST_WORDS,), jnp.float32),
            pltpu.VMEM((2, ROWS_PER_CHUNK, 512), jnp.float32),
            pltpu.VMEM((2, ROWS_PER_CHUNK, 512), jnp.int32),
            pltpu.VMEM((48, 128), jnp.float32),
            pltpu.SemaphoreType.DMA,
            pltpu.SemaphoreType.DMA,
            pltpu.SemaphoreType.DMA,
            pltpu.SemaphoreType.DMA,
        ],
    )
    h3 = sc(pred, target)
    out = pl.pallas_call(
        _tc_loss,
        grid=(N_IMG,),
        in_specs=[pl.BlockSpec((4, 48, 128), lambda i: (i, 0, 0))],
        out_specs=pl.BlockSpec((1, 1), lambda i: (0, 0)),
        out_shape=jax.ShapeDtypeStruct((1, 1), jnp.float32),
    )(h3)
    return out[0, 0]


# speedup vs baseline: 1.0832x; 1.0832x over previous
import jax
import jax.numpy as jnp
from jax import lax
from jax.experimental import pallas as pl
from jax.experimental.pallas import tpu as pltpu
from jax.experimental.pallas import tpu_sc as plsc

MBITS = 6
SHIFT = 23 - MBITS
LO = 111 << MBITS
NBINS = 24 << MBITS
NCOLS = 2 * NBINS
RSTRIDE = NCOLS + 1
NLANES = 16
HIST_WORDS = NLANES * RSTRIDE
N_IMG = 8
IMG_ELEMS = 512 * 512
NWORKERS = 32
PER_TILE = N_IMG * IMG_ELEMS // NWORKERS
CROWS = NBINS // 128

ROWS_PER_CHUNK = 16
NROWCHUNKS = 128 // ROWS_PER_CHUNK


def _sc_hist(pred_hbm, targ_hbm, out_hbm, hist, pbufs, tbufs, obuf,
             sem_p0, sem_t0, sem_p1, sem_t1):
    wid = lax.axis_index("s") * 2 + lax.axis_index("c")
    img = wid >> 2
    r0 = (wid & 3) * 128
    lane = lax.iota(jnp.int32, NLANES)
    laneoff = lane * RSTRIDE
    ones = jnp.ones((NLANES,), jnp.float32)
    zeros = jnp.zeros((NLANES,), jnp.float32)

    @plsc.parallel_loop(0, HIST_WORDS // 128, unroll=2)
    def _zero(j):
        b = j * 128
        for k in range(8):
            hist[pl.ds(b + k * NLANES, NLANES)] = zeros

    hist[pl.ds(HIST_WORDS - NLANES, NLANES)] = zeros

    sems = [[sem_p0, sem_t0], [sem_p1, sem_t1]]
    cps = {}

    def issue(c):
        slot = c & 1
        rs = r0 + c * ROWS_PER_CHUNK
        cps[c] = (
            pltpu.async_copy(pred_hbm.at[img, pl.ds(rs, ROWS_PER_CHUNK), :],
                             pbufs.at[slot], sems[slot][0]),
            pltpu.async_copy(targ_hbm.at[img, pl.ds(rs, ROWS_PER_CHUNK), :],
                             tbufs.at[slot], sems[slot][1]),
        )

    issue(0)
    for c in range(NROWCHUNKS):
        slot = c & 1
        cps[c][0].wait()
        cps[c][1].wait()
        if c + 1 < NROWCHUNKS:
            issue(c + 1)

        def body(j, carry):
            r = j >> 1
            cb = (j & 1) * 256
            for k in range(16):
                off = cb + k * NLANES
                p = pbufs[slot, r, pl.ds(off, NLANES)]
                t = tbufs[slot, r, pl.ds(off, NLANES)]
                e = jnp.where(t > 0, 1.0 - p, 1.0 + p)
                bits = lax.bitcast_convert_type(e, jnp.int32)
                b = jnp.minimum(jnp.maximum((bits >> SHIFT) - LO, 0), NBINS - 1)
                col = b + t * NBINS
                plsc.addupdate_scatter(hist, [laneoff + col], ones)
            return carry

        lax.fori_loop(0, ROWS_PER_CHUNK * 512 // 256, body, 0)

    @plsc.parallel_loop(0, NCOLS // 32, unroll=2)
    def _reduce(j):
        for k2 in range(2):
            cbase = j * 32 + k2 * NLANES
            acc = hist[pl.ds(cbase, NLANES)]
            for r in range(1, NLANES):
                acc = acc + hist[pl.ds(r * RSTRIDE + cbase, NLANES)]
            obuf[cbase >> 7, pl.ds(cbase & 127, NLANES)] = acc

    pltpu.sync_copy(obuf, out_hbm.at[wid])


def _cumsum(x, axis):
    n = x.shape[axis]
    k = 1
    while k < n:
        zshape = list(x.shape)
        zshape[axis] = k
        shifted = jnp.concatenate(
            [jnp.zeros(zshape, x.dtype), lax.slice_in_dim(x, 0, n - k, axis=axis)],
            axis=axis,
        )
        x = x + shifted
        k *= 2
    return x


def _tc_loss(hist_ref, out_ref):
    i = pl.program_id(0)
    h = jnp.sum(hist_ref[...], axis=0)
    m = h[0:CROWS, :]
    p = h[CROWS:2 * CROWS, :]
    sp = jnp.sum(p)
    sm = jnp.sum(m)
    rowm = jnp.sum(m, axis=1, keepdims=True)
    rowp = jnp.sum(p, axis=1, keepdims=True)
    pm = _cumsum(m, 1) + (_cumsum(rowm, 0) - rowm)
    pp = _cumsum(p, 1) + (_cumsum(rowp, 0) - rowp)
    a = sp - pp
    b = sm - pm
    r = lax.broadcasted_iota(jnp.int32, (CROWS, 128), 0)
    c = lax.broadcasted_iota(jnp.int32, (CROWS, 128), 1)
    vbits = ((r * 128 + c + LO) << SHIFT) + (1 << (SHIFT - 1))
    v = lax.bitcast_convert_type(vbits, jnp.float32)
    gu = (a + b + p + m) / jnp.maximum(sp + b + m, 1.0)
    gl = (a + b) / jnp.maximum(sp + b, 1.0)
    li = jnp.sum(v * (gu - gl))

    @pl.when(i == 0)
    def _():
        out_ref[...] = jnp.zeros((1, 1), jnp.float32)

    out_ref[...] = out_ref[...] + li / N_IMG


def kernel(pred, target):
    sc = pl.kernel(
        _sc_hist,
        out_type=jax.ShapeDtypeStruct((NWORKERS, 2 * CROWS, 128), jnp.float32),
        mesh=plsc.VectorSubcoreMesh(core_axis_name="c", subcore_axis_name="s"),
        compiler_params=pltpu.CompilerParams(needs_layout_passes=False),
        scratch_types=[
            pltpu.VMEM((HIST_WORDS,), jnp.float32),
            pltpu.VMEM((2, ROWS_PER_CHUNK, 512), jnp.float32),
            pltpu.VMEM((2, ROWS_PER_CHUNK, 512), jnp.int32),
            pltpu.VMEM((2 * CROWS, 128), jnp.float32),
            pltpu.SemaphoreType.DMA,
            pltpu.SemaphoreType.DMA,
            pltpu.SemaphoreType.DMA,
            pltpu.SemaphoreType.DMA,
        ],
    )
    h3 = sc(pred, target)
    out = pl.pallas_call(
        _tc_loss,
        grid=(N_IMG,),
        in_specs=[pl.BlockSpec((4, 2 * CROWS, 128), lambda i: (i, 0, 0))],
        out_specs=pl.BlockSpec((1, 1), lambda i: (0, 0)),
        out_shape=jax.ShapeDtypeStruct((1, 1), jnp.float32),
    )(h3)
    return out[0, 0]
